# Initial kernel scaffold; baseline (speedup 1.0000x reference)
import functools

import jax
import jax.numpy as jnp
from jax import lax
from jax.experimental import pallas as pl
from jax.experimental.pallas import tpu as pltpu
from jax.experimental.pallas import tpu_sc as plsc

_N = 10000
_E = 320000
_D = 128
_L = 16
_NC = 2
_NS = 16
_NW = _NC * _NS
_EPW = _E // _NW
_C = 80
_NCHUNK = _EPW // _C
_RPS = _N // _NS
_ZR = 125
_RB = 2560


def _proj_body(nodes_ref, ev_ref, wn_ref, bn_ref, we_ref, be_ref,
               vnode_ref, vedge_ref):
    @pl.when(pl.program_id(0) == 0)
    def _():
        vnode_ref[...] = (
            jnp.dot(nodes_ref[...], wn_ref[...],
                    preferred_element_type=jnp.float32) + bn_ref[...]
        )
    vedge_ref[...] = (
        jnp.dot(ev_ref[...], we_ref[...],
                preferred_element_type=jnp.float32) + be_ref[...]
    )


def _projections(nodes, edges_values, wn, bn, we, be):
    return pl.pallas_call(
        _proj_body,
        grid=(_E // _RB,),
        in_specs=[
            pl.BlockSpec((_N, _D), lambda i: (0, 0)),
            pl.BlockSpec((_RB, _D), lambda i: (i, 0)),
            pl.BlockSpec((_D, _D), lambda i: (0, 0)),
            pl.BlockSpec((1, _D), lambda i: (0, 0)),
            pl.BlockSpec((_D, _D), lambda i: (0, 0)),
            pl.BlockSpec((1, _D), lambda i: (0, 0)),
        ],
        out_specs=[
            pl.BlockSpec((_N, _D), lambda i: (0, 0)),
            pl.BlockSpec((_RB, _D), lambda i: (i, 0)),
        ],
        out_shape=[
            jax.ShapeDtypeStruct((_N, _D), jnp.float32),
            jax.ShapeDtypeStruct((_E, _D), jnp.float32),
        ],
    )(nodes, edges_values, wn, bn, we, be)


@functools.partial(
    pl.kernel,
    mesh=plsc.VectorSubcoreMesh(core_axis_name="c", subcore_axis_name="s"),
    out_type=[
        jax.ShapeDtypeStruct((_E, _D), jnp.float32),
        jax.ShapeDtypeStruct((_NC, _N, _D), jnp.float32),
    ],
    scratch_types=[
        pltpu.VMEM((_C,), jnp.int32),
        pltpu.VMEM((_C, _D), jnp.float32),
        pltpu.VMEM((_C, _D), jnp.float32),
        pltpu.VMEM((_ZR, _D), jnp.float32),
        pltpu.VMEM_SHARED((_N, _D), jnp.float32),
        pltpu.SemaphoreType.DMA,
    ],
)
def _sc_gather_scatter(vnode_hbm, dst_hbm, vedge_hbm,
                       oedge_hbm, part_hbm,
                       idx_v, gbuf, ebuf, zbuf, acc, sem):
    cid = lax.axis_index("c")
    sid = lax.axis_index("s")
    wid = cid * _NS + sid

    def _zrow(i, carry):
        for j in range(_D // _L):
            zbuf[i, pl.ds(j * _L, _L)] = jnp.zeros((_L,), jnp.float32)
        return carry

    lax.fori_loop(0, _ZR, _zrow, 0)
    row0 = sid * _RPS
    for r in range(_RPS // _ZR):
        pltpu.sync_copy(zbuf, acc.at[pl.ds(row0 + r * _ZR, _ZR)])
    plsc.subcore_barrier()

    ebase = wid * _EPW

    def _chunk(i, carry):
        off = ebase + i * _C
        pltpu.sync_copy(dst_hbm.at[pl.ds(off, _C)], idx_v)
        pltpu.async_copy(vnode_hbm.at[idx_v], gbuf, sem).wait()
        pltpu.sync_copy(vedge_hbm.at[pl.ds(off, _C)], ebuf)

        def _mrow(r2, c2):
            for j in range(_D // _L):
                sl = pl.ds(j * _L, _L)
                gbuf[r2, sl] = gbuf[r2, sl] * ebuf[r2, sl]
            return c2

        lax.fori_loop(0, _C, _mrow, 0)
        pltpu.sync_copy(gbuf, oedge_hbm.at[pl.ds(off, _C)])
        pltpu.sync_copy(gbuf, acc.at[idx_v], add=True)
        return carry

    lax.fori_loop(0, _NCHUNK, _chunk, 0)

    plsc.subcore_barrier()
    pltpu.sync_copy(acc.at[pl.ds(row0, _RPS)],
                    part_hbm.at[cid, pl.ds(row0, _RPS)])


def _add_body(p_ref, o_ref):
    o_ref[...] = p_ref[0] + p_ref[1]


def _final_add(partials):
    return pl.pallas_call(
        _add_body,
        out_shape=jax.ShapeDtypeStruct((_N, _D), jnp.float32),
    )(partials)


def kernel(nodes, edges_index, edges_values,
           WQ_node_w, WQ_node_b, WQ_edge_w, WQ_edge_b,
           WK_node_w, WK_node_b, WK_edge_w, WK_edge_b,
           WV_node_w, WV_node_b, WV_edge_w, WV_edge_b):
    dst = edges_index[1].astype(jnp.int32)
    wn = WV_node_w.T
    we = WV_edge_w.T
    bn = WV_node_b.reshape(1, _D)
    be = WV_edge_b.reshape(1, _D)
    v_node, v_edge = _projections(nodes, edges_values, wn, bn, we, be)
    output_edges, partials = _sc_gather_scatter(v_node, dst, v_edge)
    output_nodes = _final_add(partials)
    return (output_nodes, output_edges)


# speedup vs baseline: 5.4293x; 5.4293x over previous
import functools

import jax
import jax.numpy as jnp
from jax import lax
from jax.experimental import pallas as pl
from jax.experimental.pallas import tpu as pltpu
from jax.experimental.pallas import tpu_sc as plsc

_N = 10000
_E = 320000
_D = 128
_L = 16
_NC = 2
_NS = 16
_NW = _NC * _NS
_EPW = _E // _NW
_C = 80
_NCHUNK = _EPW // _C
_NA = 10240
_RPS = _NA // _NS
_ZR = 128
_RB = 2560


def _proj_body(nodes_ref, ev_ref, wn_ref, bn_ref, we_ref, be_ref,
               vnode_ref, vedge_ref):
    @pl.when(pl.program_id(0) == 0)
    def _():
        vnode_ref[...] = (
            jnp.dot(nodes_ref[...], wn_ref[...],
                    preferred_element_type=jnp.float32) + bn_ref[...]
        )
    vedge_ref[...] = (
        jnp.dot(ev_ref[...], we_ref[...],
                preferred_element_type=jnp.float32) + be_ref[...]
    )


def _projections(nodes, edges_values, wn, bn, we, be):
    return pl.pallas_call(
        _proj_body,
        grid=(_E // _RB,),
        in_specs=[
            pl.BlockSpec((_N, _D), lambda i: (0, 0)),
            pl.BlockSpec((_RB, _D), lambda i: (i, 0)),
            pl.BlockSpec((_D, _D), lambda i: (0, 0)),
            pl.BlockSpec((1, _D), lambda i: (0, 0)),
            pl.BlockSpec((_D, _D), lambda i: (0, 0)),
            pl.BlockSpec((1, _D), lambda i: (0, 0)),
        ],
        out_specs=[
            pl.BlockSpec((_N, _D), lambda i: (0, 0)),
            pl.BlockSpec((_RB, _D), lambda i: (i, 0)),
        ],
        out_shape=[
            jax.ShapeDtypeStruct((_N, _D), jnp.float32),
            jax.ShapeDtypeStruct((_E, _D), jnp.float32),
        ],
    )(nodes, edges_values, wn, bn, we, be)


@functools.partial(
    pl.kernel,
    mesh=plsc.VectorSubcoreMesh(core_axis_name="c", subcore_axis_name="s"),
    out_type=[
        jax.ShapeDtypeStruct((_E, _D), jnp.float32),
        jax.ShapeDtypeStruct((_NC, _NA, _D), jnp.float32),
    ],
    scratch_types=[
        pltpu.VMEM((_C,), jnp.int32),
        pltpu.VMEM((_C, _D), jnp.float32),
        pltpu.VMEM((_C, _D), jnp.float32),
        pltpu.VMEM((_ZR, _D), jnp.float32),
        pltpu.VMEM_SHARED((_NA, _D), jnp.float32),
        pltpu.SemaphoreType.DMA,
    ],
)
def _sc_gather_scatter(vnode_hbm, dst_hbm, vedge_hbm,
                       oedge_hbm, part_hbm,
                       idx_v, gbuf, ebuf, zbuf, acc, sem):
    cid = lax.axis_index("c")
    sid = lax.axis_index("s")
    wid = cid * _NS + sid

    def _zrow(i, carry):
        for j in range(_D // _L):
            zbuf[i, pl.ds(j * _L, _L)] = jnp.zeros((_L,), jnp.float32)
        return carry

    lax.fori_loop(0, _ZR, _zrow, 0)
    row0 = sid * _RPS
    for r in range(_RPS // _ZR):
        pltpu.sync_copy(zbuf, acc.at[pl.ds(row0 + r * _ZR, _ZR)])
    plsc.subcore_barrier()

    ebase = wid * _EPW

    def _chunk(i, carry):
        off = ebase + i * _C
        pltpu.sync_copy(dst_hbm.at[pl.ds(off, _C)], idx_v)
        pltpu.async_copy(vnode_hbm.at[idx_v], gbuf, sem).wait()
        pltpu.sync_copy(vedge_hbm.at[pl.ds(off, _C)], ebuf)

        def _mrow(r2, c2):
            for j in range(_D // _L):
                sl = pl.ds(j * _L, _L)
                gbuf[r2, sl] = gbuf[r2, sl] * ebuf[r2, sl]
            return c2

        lax.fori_loop(0, _C, _mrow, 0)
        pltpu.sync_copy(gbuf, oedge_hbm.at[pl.ds(off, _C)])
        pltpu.sync_copy(gbuf, acc.at[idx_v], add=True)
        return carry

    lax.fori_loop(0, _NCHUNK, _chunk, 0)

    plsc.subcore_barrier()
    pltpu.sync_copy(acc.at[pl.ds(row0, _RPS)],
                    part_hbm.at[cid, pl.ds(row0, _RPS)])


def _add_body(p_ref, o_ref):
    o_ref[...] = p_ref[0, :_N, :] + p_ref[1, :_N, :]


def _final_add(partials):
    return pl.pallas_call(
        _add_body,
        out_shape=jax.ShapeDtypeStruct((_N, _D), jnp.float32),
    )(partials)


def kernel(nodes, edges_index, edges_values,
           WQ_node_w, WQ_node_b, WQ_edge_w, WQ_edge_b,
           WK_node_w, WK_node_b, WK_edge_w, WK_edge_b,
           WV_node_w, WV_node_b, WV_edge_w, WV_edge_b):
    dst = edges_index[1].astype(jnp.int32)
    wn = WV_node_w.T
    we = WV_edge_w.T
    bn = WV_node_b.reshape(1, _D)
    be = WV_edge_b.reshape(1, _D)
    v_node, v_edge = _projections(nodes, edges_values, wn, bn, we, be)
    output_edges, partials = _sc_gather_scatter(v_node, dst, v_edge)
    output_nodes = _final_add(partials)
    return (output_nodes, output_edges)


# speedup vs baseline: 5.5213x; 1.0169x over previous
import functools

import jax
import jax.numpy as jnp
from jax import lax
from jax.experimental import pallas as pl
from jax.experimental.pallas import tpu as pltpu
from jax.experimental.pallas import tpu_sc as plsc

_N = 10000
_E = 320000
_D = 128
_L = 16
_NC = 2
_NS = 16
_NW = _NC * _NS
_EPW = _E // _NW
_C = 80
_NCHUNK = _EPW // _C
_GS = 8
_G = 16
_NA = 10240
_RPS = _NA // _NS
_RB = 2560


def _proj_body(nodes_ref, ev_ref, wn_ref, bn_ref, we_ref, be_ref,
               vnode_ref, vedge_ref):
    @pl.when(pl.program_id(0) == 0)
    def _():
        vnode_ref[...] = (
            jnp.dot(nodes_ref[...], wn_ref[...],
                    preferred_element_type=jnp.float32) + bn_ref[...]
        )
    vedge_ref[...] = (
        jnp.dot(ev_ref[...], we_ref[...],
                preferred_element_type=jnp.float32) + be_ref[...]
    )


def _projections(nodes, edges_values, wn, bn, we, be):
    return pl.pallas_call(
        _proj_body,
        grid=(_E // _RB,),
        in_specs=[
            pl.BlockSpec((_N, _D), lambda i: (0, 0)),
            pl.BlockSpec((_RB, _D), lambda i: (i, 0)),
            pl.BlockSpec((_D, _D), lambda i: (0, 0)),
            pl.BlockSpec((1, _D), lambda i: (0, 0)),
            pl.BlockSpec((_D, _D), lambda i: (0, 0)),
            pl.BlockSpec((1, _D), lambda i: (0, 0)),
        ],
        out_specs=[
            pl.BlockSpec((_N, _D), lambda i: (0, 0)),
            pl.BlockSpec((_RB, _D), lambda i: (i, 0)),
        ],
        out_shape=[
            jax.ShapeDtypeStruct((_N, _D), jnp.float32),
            jax.ShapeDtypeStruct((_E, _D), jnp.float32),
        ],
    )(nodes, edges_values, wn, bn, we, be)


@functools.partial(
    pl.kernel,
    mesh=plsc.VectorSubcoreMesh(core_axis_name="c", subcore_axis_name="s"),
    out_type=[
        jax.ShapeDtypeStruct((_E, _D), jnp.float32),
        jax.ShapeDtypeStruct((_NC, _NA, _D), jnp.float32),
    ],
    scratch_types=[
        pltpu.VMEM((2, _GS, _C), jnp.int32),
        pltpu.VMEM((2, _C, _D), jnp.float32),
        pltpu.VMEM((2, _C, _D), jnp.float32),
        pltpu.VMEM_SHARED((_NA, _D), jnp.float32),
        pltpu.SemaphoreType.DMA((2,)),
        pltpu.SemaphoreType.DMA((2,)),
        pltpu.SemaphoreType.DMA((2,)),
        pltpu.SemaphoreType.DMA((2,)),
        pltpu.SemaphoreType.DMA((2,)),
    ],
)
def _sc_gather_scatter(vnode_hbm, dst_hbm, vedge_hbm,
                       oedge_hbm, part_hbm,
                       idx, gbuf, ebuf, acc,
                       sem_g, sem_e, sem_o, sem_s, sem_i):
    cid = lax.axis_index("c")
    sid = lax.axis_index("s")
    wid = cid * _NS + sid
    ebase = wid * _EPW

    for sl in range(2):
        def _zrow(i, carry, _sl=sl):
            for j in range(_D // _L):
                gbuf[_sl, i, pl.ds(j * _L, _L)] = jnp.zeros((_L,), jnp.float32)
            return carry

        lax.fori_loop(0, _C, _zrow, 0)
    row0 = sid * _RPS
    for r in range(_RPS // _C):
        pltpu.sync_copy(gbuf.at[r % 2], acc.at[pl.ds(row0 + r * _C, _C)])
    plsc.subcore_barrier()

    def _start_idx(s, g):
        pltpu.async_copy(dst_hbm.at[wid, g], idx.at[s], sem_i.at[s])

    def _wait_idx(s, g):
        pltpu.make_async_copy(dst_hbm.at[wid, g], idx.at[s],
                              sem_i.at[s]).wait()

    def _start_in(s, i, gs, j):
        pltpu.async_copy(vnode_hbm.at[idx.at[gs, j]], gbuf.at[s],
                         sem_g.at[s])
        pltpu.async_copy(vedge_hbm.at[pl.ds(ebase + i * _C, _C)],
                         ebuf.at[s], sem_e.at[s])

    def _wait_in(s, i, gs, j):
        pltpu.make_async_copy(vnode_hbm.at[idx.at[gs, j]], gbuf.at[s],
                              sem_g.at[s]).wait()
        pltpu.make_async_copy(vedge_hbm.at[pl.ds(ebase + i * _C, _C)],
                              ebuf.at[s], sem_e.at[s]).wait()

    def _mul(s):
        def _mrow(r2, c2):
            for u in range(2):
                for j in range(_D // _L):
                    sl = pl.ds(j * _L, _L)
                    gbuf[s, 2 * r2 + u, sl] = (
                        gbuf[s, 2 * r2 + u, sl] * ebuf[s, 2 * r2 + u, sl])
            return c2

        lax.fori_loop(0, _C // 2, _mrow, 0)

    def _start_out(s, i, gs, j):
        pltpu.async_copy(gbuf.at[s],
                         oedge_hbm.at[pl.ds(ebase + i * _C, _C)], sem_o.at[s])
        pltpu.async_copy(gbuf.at[s], acc.at[idx.at[gs, j]], sem_s.at[s],
                         add=True)

    def _wait_out(s, i, gs, j):
        pltpu.make_async_copy(gbuf.at[s],
                              oedge_hbm.at[pl.ds(ebase + i * _C, _C)],
                              sem_o.at[s]).wait()
        pltpu.make_async_copy(gbuf.at[s], acc.at[idx.at[gs, j]],
                              sem_s.at[s]).wait()

    pltpu.sync_copy(dst_hbm.at[wid, 0], idx.at[0])
    _start_in(0, 0, 0, 0)

    def _body(i, carry):
        s = lax.rem(i, 2)
        ns = 1 - s
        g = lax.div(i, _GS)
        j = lax.rem(i, _GS)
        gs = lax.rem(g, 2)
        ngs = 1 - gs

        @pl.when(i >= 1)
        def _():
            _wait_out(ns, i - 1, lax.rem(lax.div(i - 1, _GS), 2),
                      lax.rem(i - 1, _GS))

        @pl.when(jnp.logical_and(j == 1, g < _G - 1))
        def _():
            _start_idx(ngs, g + 1)

        @pl.when(j == _GS - 1)
        def _():
            _wait_idx(ngs, g + 1)

        nxt_gs = lax.rem(lax.div(i + 1, _GS), 2)
        _start_in(ns, i + 1, nxt_gs, lax.rem(i + 1, _GS))
        _wait_in(s, i, gs, j)
        _mul(s)
        _start_out(s, i, gs, j)
        return carry

    lax.fori_loop(0, _NCHUNK - 1, _body, 0)
    last = _NCHUNK - 1
    s_last = last % 2
    g_last = (last // _GS) % 2
    j_last = last % _GS
    _wait_in(s_last, last, g_last, j_last)
    _mul(s_last)
    _start_out(s_last, last, g_last, j_last)
    _wait_out(1 - s_last, last - 1, ((last - 1) // _GS) % 2, (last - 1) % _GS)
    _wait_out(s_last, last, g_last, j_last)

    plsc.subcore_barrier()
    pltpu.sync_copy(acc.at[pl.ds(row0, _RPS)],
                    part_hbm.at[cid, pl.ds(row0, _RPS)])


def _add_body(p_ref, o_ref):
    o_ref[...] = p_ref[0, :_N, :] + p_ref[1, :_N, :]


def _final_add(partials):
    return pl.pallas_call(
        _add_body,
        out_shape=jax.ShapeDtypeStruct((_N, _D), jnp.float32),
    )(partials)


def kernel(nodes, edges_index, edges_values,
           WQ_node_w, WQ_node_b, WQ_edge_w, WQ_edge_b,
           WK_node_w, WK_node_b, WK_edge_w, WK_edge_b,
           WV_node_w, WV_node_b, WV_edge_w, WV_edge_b):
    dst = edges_index[1].astype(jnp.int32).reshape(_NW, _EPW)
    dst = jnp.pad(dst, ((0, 0), (0, _G * _GS * _C - _EPW)))
    dst = dst.reshape(_NW, _G, _GS, _C)
    wn = WV_node_w.T
    we = WV_edge_w.T
    bn = WV_node_b.reshape(1, _D)
    be = WV_edge_b.reshape(1, _D)
    v_node, v_edge = _projections(nodes, edges_values, wn, bn, we, be)
    output_edges, partials = _sc_gather_scatter(v_node, dst, v_edge)
    output_nodes = _final_add(partials)
    return (output_nodes, output_edges)


# speedup vs baseline: 9.9812x; 1.8078x over previous
import functools

import jax
import jax.numpy as jnp
from jax import lax
from jax.experimental import pallas as pl
from jax.experimental.pallas import tpu as pltpu
from jax.experimental.pallas import tpu_sc as plsc

_N = 10000
_E = 320000
_D = 128
_L = 16
_NC = 2
_NS = 16
_NW = _NC * _NS
_EPW = _E // _NW
_C = 80
_NCHUNK = _EPW // _C
_GS = 8
_G = 16
_NA = 10240
_RPS = _NA // _NS
_RB = 2560


def _proj_body(nodes_ref, ev_ref, wn_ref, bn_ref, we_ref, be_ref,
               vnode_ref, vedge_ref):
    @pl.when(pl.program_id(0) == 0)
    def _():
        vnode_ref[...] = (
            jnp.dot(nodes_ref[...], wn_ref[...],
                    preferred_element_type=jnp.float32) + bn_ref[...]
        )
    vedge_ref[...] = (
        jnp.dot(ev_ref[...], we_ref[...],
                preferred_element_type=jnp.float32) + be_ref[...]
    )


def _projections(nodes, edges_values, wn, bn, we, be):
    return pl.pallas_call(
        _proj_body,
        grid=(_E // _RB,),
        in_specs=[
            pl.BlockSpec((_N, _D), lambda i: (0, 0)),
            pl.BlockSpec((_RB, _D), lambda i: (i, 0)),
            pl.BlockSpec((_D, _D), lambda i: (0, 0)),
            pl.BlockSpec((1, _D), lambda i: (0, 0)),
            pl.BlockSpec((_D, _D), lambda i: (0, 0)),
            pl.BlockSpec((1, _D), lambda i: (0, 0)),
        ],
        out_specs=[
            pl.BlockSpec((_N, _D), lambda i: (0, 0)),
            pl.BlockSpec((_RB, _D), lambda i: (i, 0)),
        ],
        out_shape=[
            jax.ShapeDtypeStruct((_N, _D), jnp.float32),
            jax.ShapeDtypeStruct((_E, _D), jnp.float32),
        ],
    )(nodes, edges_values, wn, bn, we, be)


@functools.partial(
    pl.kernel,
    mesh=plsc.VectorSubcoreMesh(core_axis_name="c", subcore_axis_name="s"),
    out_type=[
        jax.ShapeDtypeStruct((_E, _D), jnp.float32),
        jax.ShapeDtypeStruct((_NC, _NA, _D), jnp.float32),
    ],
    scratch_types=[
        pltpu.VMEM((2, _GS, _C), jnp.int32),
        pltpu.VMEM((2, _C, _D), jnp.float32),
        pltpu.VMEM((2, _C, _D), jnp.float32),
        pltpu.VMEM_SHARED((_NA, _D), jnp.float32),
        pltpu.SemaphoreType.DMA((2,)),
        pltpu.SemaphoreType.DMA((2,)),
        pltpu.SemaphoreType.DMA((2,)),
        pltpu.SemaphoreType.DMA((2,)),
        pltpu.SemaphoreType.DMA((2,)),
    ],
)
def _sc_gather_scatter(vnode_hbm, dst_hbm, vedge_hbm,
                       oedge_hbm, part_hbm,
                       idx, gbuf, ebuf, acc,
                       sem_g, sem_e, sem_o, sem_s, sem_i):
    cid = lax.axis_index("c")
    sid = lax.axis_index("s")
    wid = cid * _NS + sid
    ebase = wid * _EPW

    for sl in range(2):
        def _zrow(i, carry, _sl=sl):
            for j in range(_D // _L):
                gbuf[_sl, i, pl.ds(j * _L, _L)] = jnp.zeros((_L,), jnp.float32)
            return carry

        lax.fori_loop(0, _C, _zrow, 0)
    row0 = sid * _RPS
    for r in range(_RPS // _C):
        pltpu.sync_copy(gbuf.at[r % 2], acc.at[pl.ds(row0 + r * _C, _C)])
    plsc.subcore_barrier()

    def _start_idx(s, g):
        pltpu.async_copy(dst_hbm.at[wid, g], idx.at[s], sem_i.at[s])

    def _wait_idx(s, g):
        pltpu.make_async_copy(dst_hbm.at[wid, g], idx.at[s],
                              sem_i.at[s]).wait()

    def _start_in(s, i, gs, j):
        pltpu.async_copy(vnode_hbm.at[idx.at[gs, j]], gbuf.at[s],
                         sem_g.at[s])
        pltpu.async_copy(vedge_hbm.at[pl.ds(ebase + i * _C, _C)],
                         ebuf.at[s], sem_e.at[s])

    def _wait_in(s, i, gs, j):
        pltpu.make_async_copy(vnode_hbm.at[idx.at[gs, j]], gbuf.at[s],
                              sem_g.at[s]).wait()
        pltpu.make_async_copy(vedge_hbm.at[pl.ds(ebase + i * _C, _C)],
                              ebuf.at[s], sem_e.at[s]).wait()

    def _mul(s):
        @plsc.parallel_loop(0, _C, step=1, unroll=4)
        def _mrow(r2):
            for j in range(_D // _L):
                sl = pl.ds(j * _L, _L)
                gbuf[s, r2, sl] = gbuf[s, r2, sl] * ebuf[s, r2, sl]

    def _start_out(s, i, gs, j):
        pltpu.async_copy(gbuf.at[s],
                         oedge_hbm.at[pl.ds(ebase + i * _C, _C)], sem_o.at[s])
        pltpu.async_copy(gbuf.at[s], acc.at[idx.at[gs, j]], sem_s.at[s],
                         add=True)

    def _wait_out(s, i, gs, j):
        pltpu.make_async_copy(gbuf.at[s],
                              oedge_hbm.at[pl.ds(ebase + i * _C, _C)],
                              sem_o.at[s]).wait()
        pltpu.make_async_copy(gbuf.at[s], acc.at[idx.at[gs, j]],
                              sem_s.at[s]).wait()

    pltpu.sync_copy(dst_hbm.at[wid, 0], idx.at[0])
    _start_in(0, 0, 0, 0)

    def _body(i, carry):
        s = lax.rem(i, 2)
        ns = 1 - s
        g = lax.div(i, _GS)
        j = lax.rem(i, _GS)
        gs = lax.rem(g, 2)
        ngs = 1 - gs

        @pl.when(i >= 1)
        def _():
            _wait_out(ns, i - 1, lax.rem(lax.div(i - 1, _GS), 2),
                      lax.rem(i - 1, _GS))

        @pl.when(jnp.logical_and(j == 1, g < _G - 1))
        def _():
            _start_idx(ngs, g + 1)

        @pl.when(j == _GS - 1)
        def _():
            _wait_idx(ngs, g + 1)

        nxt_gs = lax.rem(lax.div(i + 1, _GS), 2)
        _start_in(ns, i + 1, nxt_gs, lax.rem(i + 1, _GS))
        _wait_in(s, i, gs, j)
        _mul(s)
        _start_out(s, i, gs, j)
        return carry

    lax.fori_loop(0, _NCHUNK - 1, _body, 0)
    last = _NCHUNK - 1
    s_last = last % 2
    g_last = (last // _GS) % 2
    j_last = last % _GS
    _wait_in(s_last, last, g_last, j_last)
    _mul(s_last)
    _start_out(s_last, last, g_last, j_last)
    _wait_out(1 - s_last, last - 1, ((last - 1) // _GS) % 2, (last - 1) % _GS)
    _wait_out(s_last, last, g_last, j_last)

    plsc.subcore_barrier()
    pltpu.sync_copy(acc.at[pl.ds(row0, _RPS)],
                    part_hbm.at[cid, pl.ds(row0, _RPS)])


def _add_body(p_ref, o_ref):
    o_ref[...] = p_ref[0, :_N, :] + p_ref[1, :_N, :]


def _final_add(partials):
    return pl.pallas_call(
        _add_body,
        out_shape=jax.ShapeDtypeStruct((_N, _D), jnp.float32),
    )(partials)


def kernel(nodes, edges_index, edges_values,
           WQ_node_w, WQ_node_b, WQ_edge_w, WQ_edge_b,
           WK_node_w, WK_node_b, WK_edge_w, WK_edge_b,
           WV_node_w, WV_node_b, WV_edge_w, WV_edge_b):
    dst = edges_index[1].astype(jnp.int32).reshape(_NW, _EPW)
    dst = jnp.pad(dst, ((0, 0), (0, _G * _GS * _C - _EPW)))
    dst = dst.reshape(_NW, _G, _GS, _C)
    wn = WV_node_w.T
    we = WV_edge_w.T
    bn = WV_node_b.reshape(1, _D)
    be = WV_edge_b.reshape(1, _D)
    v_node, v_edge = _projections(nodes, edges_values, wn, bn, we, be)
    output_edges, partials = _sc_gather_scatter(v_node, dst, v_edge)
    output_nodes = _final_add(partials)
    return (output_nodes, output_edges)
